# baseline (speedup 1.0000x reference)
import functools

import jax
import jax.numpy as jnp
from jax import lax
from jax.experimental import pallas as pl
from jax.experimental.pallas import tpu as pltpu
from jax.experimental.pallas import tpu_sc as plsc

NC = 2
NS = 16
NW = NC * NS

DIM = 64
CHUNK = 512


def _make_gather(n_rows: int):
    assert n_rows % (NW * CHUNK) == 0
    n_chunks = n_rows // (NW * CHUNK)
    mesh = plsc.VectorSubcoreMesh(core_axis_name="c", subcore_axis_name="s")

    @functools.partial(
        pl.kernel,
        mesh=mesh,
        out_type=jax.ShapeDtypeStruct((NW, n_chunks, CHUNK, DIM), jnp.float32),
        scratch_types=[
            pltpu.VMEM((n_chunks, CHUNK), jnp.int32),
            pltpu.VMEM((CHUNK, DIM), jnp.float32),
            pltpu.VMEM((CHUNK, DIM), jnp.float32),
            pltpu.SemaphoreType.DMA,
            pltpu.SemaphoreType.DMA,
        ],
        compiler_params=pltpu.CompilerParams(use_tc_tiling_on_sc=False),
    )
    def gather(table_hbm, idx_hbm, out_hbm, idx_v, rows0, rows1, sem0, sem1):
        wid = lax.axis_index("s") * NC + lax.axis_index("c")
        pltpu.sync_copy(idx_hbm.at[wid], idx_v)

        rows = (rows0, rows1)
        sems = (sem0, sem1)
        copies = [None, None]
        copies[0] = pltpu.async_copy(table_hbm.at[idx_v.at[0]], rows[0], sems[0])
        for i in range(n_chunks):
            buf = i % 2
            nxt = (i + 1) % 2
            if i + 1 < n_chunks:
                copies[nxt] = pltpu.async_copy(
                    table_hbm.at[idx_v.at[i + 1]], rows[nxt], sems[nxt]
                )
            copies[buf].wait()
            pltpu.sync_copy(rows[buf], out_hbm.at[wid, i])

    return gather


def kernel(indices, embs):
    batch, hist = indices.shape
    n_rows = batch * hist
    n_chunks = n_rows // (NW * CHUNK)
    idx = indices.reshape(NW, n_chunks, CHUNK)
    out = _make_gather(n_rows)(embs, idx)
    return out.reshape(batch, hist, DIM)
